# baseline (speedup 1.0000x reference)
import functools
import jax
import jax.numpy as jnp
from jax import lax
from jax.experimental import pallas as pl
from jax.experimental.pallas import tpu as pltpu
from jax.experimental.pallas import tpu_sc as plsc

N, D = 16384, 1024
SC_N = 2048
TC_N = N - SC_N
NW = 32
ROWS_PER_W = SC_N // NW
GROUP = 8
NGROUPS = ROWS_PER_W // GROUP
NBUF = 2
NCHUNK = D // 16
UNROLL = 4
BN = 1024
TEMP = 0.1
MARGIN = 1.0

_mesh = plsc.VectorSubcoreMesh(
    core_axis_name="c", subcore_axis_name="s", num_cores=2, num_subcores=16
)


@functools.partial(
    pl.kernel,
    out_type=jax.ShapeDtypeStruct((SC_N, 80), jnp.float32),
    mesh=_mesh,
    scratch_types=[
        pltpu.VMEM((NBUF, 6, GROUP, D), jnp.float32),
        pltpu.VMEM((GROUP, 80), jnp.float32),
        pltpu.SemaphoreType.DMA,
        pltpu.SemaphoreType.DMA,
    ],
)
def _sc_sums(ar, ai, pr, pi, nr, ni, out, buf, stage, sem0, sem1):
    sems = (sem0, sem1)
    wid = lax.axis_index("s") * 2 + lax.axis_index("c")
    base = TC_N + wid * ROWS_PER_W
    inputs = (ar, ai, pr, pi, nr, ni)

    def issue(g, b):
        r0 = base + g * GROUP
        for k in range(6):
            pltpu.async_copy(inputs[k].at[pl.ds(r0, GROUP), :], buf.at[b, k], sems[b])

    def wait_group(g, b):
        r0 = base + g * GROUP
        for k in range(6):
            pltpu.make_async_copy(
                inputs[k].at[pl.ds(r0, GROUP), :], buf.at[b, k], sems[b]
            ).wait()

    def compute_group(g, b):
        for r in range(GROUP):
            def col_body(j, accs):
                accs = list(accs)
                for u in range(UNROLL):
                    sl = pl.ds((j * UNROLL + u) * 16, 16)
                    v_ar = buf[b, 0, r, sl]
                    v_ai = buf[b, 1, r, sl]
                    v_pr = buf[b, 2, r, sl]
                    v_pi = buf[b, 3, r, sl]
                    v_nr = buf[b, 4, r, sl]
                    v_ni = buf[b, 5, r, sl]
                    prods = (v_ar * v_pr, v_ai * v_pi,
                             v_ar * v_nr, v_ai * v_ni,
                             v_ar * v_ar, v_ai * v_ai,
                             v_pr * v_pr, v_pi * v_pi,
                             v_nr * v_nr, v_ni * v_ni)
                    for q in range(10):
                        accs[q] = accs[q] + prods[q]
                return tuple(accs)

            z = jnp.zeros((16,), jnp.float32)
            accs = lax.fori_loop(0, NCHUNK // UNROLL, col_body, (z,) * 10)
            for q in range(5):
                stage[r, pl.ds(q * 16, 16)] = accs[2 * q] + accs[2 * q + 1]
        r0 = wid * ROWS_PER_W + g * GROUP
        pltpu.sync_copy(stage, out.at[pl.ds(r0, GROUP), :])

    issue(0, 0)
    issue(1, 1)

    @pl.loop(0, NGROUPS, step=NBUF)
    def _(g0):
        for b in range(NBUF):
            g = g0 + b
            wait_group(g, b)
            compute_group(g, b)

            @pl.when(g + NBUF < NGROUPS)
            def _():
                issue(g + NBUF, b)


def _row_loss(dot_p, dot_n, ssa, ssp, ssn):
    mag_a = jnp.sqrt(ssa + 1e-8)
    mag_p = jnp.sqrt(ssp + 1e-8)
    mag_n = jnp.sqrt(ssn + 1e-8)
    pos = dot_p / (mag_a * mag_p + 1e-8)
    neg = dot_n / (mag_a * mag_n + 1e-8)
    t = (neg - pos) / TEMP
    softplus = jnp.maximum(t, 0.0) + jnp.log1p(jnp.exp(-jnp.abs(t)))
    sep = jnp.maximum(neg + MARGIN, 0.0)
    return softplus + sep


def _tc_body(ar, ai, pr, pi, nr, ni, out_ref):
    step = pl.program_id(0)
    a_r = ar[...]
    a_i = ai[...]
    p_r = pr[...]
    p_i = pi[...]
    n_r = nr[...]
    n_i = ni[...]

    dot_p = jnp.sum(a_r * p_r + a_i * p_i, axis=1)
    dot_n = jnp.sum(a_r * n_r + a_i * n_i, axis=1)
    ssa = jnp.sum(a_r * a_r + a_i * a_i, axis=1)
    ssp = jnp.sum(p_r * p_r + p_i * p_i, axis=1)
    ssn = jnp.sum(n_r * n_r + n_i * n_i, axis=1)

    part = jnp.sum(_row_loss(dot_p, dot_n, ssa, ssp, ssn))

    @pl.when(step == 0)
    def _():
        out_ref[0, 0] = 0.0

    out_ref[0, 0] += part


def _tail_body(x_ref, tc_ref, out_ref):
    x = x_ref[...]
    s_p = jnp.sum(x[:, 0:16], axis=1)
    s_n = jnp.sum(x[:, 16:32], axis=1)
    s_a = jnp.sum(x[:, 32:48], axis=1)
    s_pp = jnp.sum(x[:, 48:64], axis=1)
    s_nn = jnp.sum(x[:, 64:80], axis=1)
    sc_sum = jnp.sum(_row_loss(s_p, s_n, s_a, s_pp, s_nn))
    out_ref[0, 0] = (sc_sum + tc_ref[0, 0]) * (1.0 / N)


def kernel(anchor_real, anchor_imag, positive_real, positive_imag, negative_real, negative_imag):
    full = (anchor_real, anchor_imag, positive_real, positive_imag,
            negative_real, negative_imag)

    sc_sums = _sc_sums(*full)

    tc_spec = pl.BlockSpec((BN, D), lambda i: (i, 0))
    tc_part = pl.pallas_call(
        _tc_body,
        grid=(TC_N // BN,),
        in_specs=[tc_spec] * 6,
        out_specs=pl.BlockSpec(memory_space=pltpu.SMEM),
        out_shape=jax.ShapeDtypeStruct((1, 1), jnp.float32),
        compiler_params=pltpu.CompilerParams(
            dimension_semantics=("arbitrary",),
        ),
    )(*full)

    out = pl.pallas_call(
        _tail_body,
        in_specs=[
            pl.BlockSpec((SC_N, 80), lambda: (0, 0)),
            pl.BlockSpec(memory_space=pltpu.SMEM),
        ],
        out_specs=pl.BlockSpec(memory_space=pltpu.SMEM),
        out_shape=jax.ShapeDtypeStruct((1, 1), jnp.float32),
    )(sc_sums, tc_part)
    return out[0, 0]


# speedup vs baseline: 1.0117x; 1.0117x over previous
import functools
import jax
import jax.numpy as jnp
from jax import lax
from jax.experimental import pallas as pl
from jax.experimental.pallas import tpu as pltpu
from jax.experimental.pallas import tpu_sc as plsc

N, D = 16384, 1024
SC_N = 2048
TC_N = N - SC_N
NW = 32
ROWS_PER_W = SC_N // NW
GROUP = 4
NGROUPS = ROWS_PER_W // GROUP
NBUF = 2
NCHUNK = D // 16
UNROLL = 2
BN = 512
TEMP = 0.1
MARGIN = 1.0

_mesh = plsc.VectorSubcoreMesh(
    core_axis_name="c", subcore_axis_name="s", num_cores=2, num_subcores=16
)


@functools.partial(
    pl.kernel,
    out_type=jax.ShapeDtypeStruct((SC_N, 80), jnp.float32),
    mesh=_mesh,
    scratch_types=[
        pltpu.VMEM((NBUF, 6, GROUP, D), jnp.float32),
        pltpu.VMEM((GROUP, 80), jnp.float32),
        pltpu.SemaphoreType.DMA,
        pltpu.SemaphoreType.DMA,
    ],
)
def _sc_sums(ar, ai, pr, pi, nr, ni, out, buf, stage, sem0, sem1):
    sems = (sem0, sem1)
    wid = lax.axis_index("s") * 2 + lax.axis_index("c")
    base = TC_N + wid * ROWS_PER_W
    inputs = (ar, ai, pr, pi, nr, ni)

    def issue(g, b):
        r0 = base + g * GROUP
        for k in range(6):
            pltpu.async_copy(inputs[k].at[pl.ds(r0, GROUP), :], buf.at[b, k], sems[b])

    def wait_group(g, b):
        r0 = base + g * GROUP
        for k in range(6):
            pltpu.make_async_copy(
                inputs[k].at[pl.ds(r0, GROUP), :], buf.at[b, k], sems[b]
            ).wait()

    def compute_group(g, b):
        for r in range(GROUP):
            def col_body(j, accs):
                accs = list(accs)
                for u in range(UNROLL):
                    sl = pl.ds((j * UNROLL + u) * 16, 16)
                    v_ar = buf[b, 0, r, sl]
                    v_ai = buf[b, 1, r, sl]
                    v_pr = buf[b, 2, r, sl]
                    v_pi = buf[b, 3, r, sl]
                    v_nr = buf[b, 4, r, sl]
                    v_ni = buf[b, 5, r, sl]
                    prods = (v_ar * v_pr, v_ai * v_pi,
                             v_ar * v_nr, v_ai * v_ni,
                             v_ar * v_ar, v_ai * v_ai,
                             v_pr * v_pr, v_pi * v_pi,
                             v_nr * v_nr, v_ni * v_ni)
                    for q in range(10):
                        accs[q] = accs[q] + prods[q]
                return tuple(accs)

            z = jnp.zeros((16,), jnp.float32)
            accs = lax.fori_loop(0, NCHUNK // UNROLL, col_body, (z,) * 10)
            for q in range(5):
                stage[r, pl.ds(q * 16, 16)] = accs[2 * q] + accs[2 * q + 1]
        r0 = wid * ROWS_PER_W + g * GROUP
        pltpu.sync_copy(stage, out.at[pl.ds(r0, GROUP), :])

    issue(0, 0)
    issue(1, 1)

    @pl.loop(0, NGROUPS, step=NBUF)
    def _(g0):
        for b in range(NBUF):
            g = g0 + b
            wait_group(g, b)
            compute_group(g, b)

            @pl.when(g + NBUF < NGROUPS)
            def _():
                issue(g + NBUF, b)


def _row_loss(dot_p, dot_n, ssa, ssp, ssn):
    mag_a = jnp.sqrt(ssa + 1e-8)
    mag_p = jnp.sqrt(ssp + 1e-8)
    mag_n = jnp.sqrt(ssn + 1e-8)
    pos = dot_p / (mag_a * mag_p + 1e-8)
    neg = dot_n / (mag_a * mag_n + 1e-8)
    t = (neg - pos) / TEMP
    softplus = jnp.maximum(t, 0.0) + jnp.log1p(jnp.exp(-jnp.abs(t)))
    sep = jnp.maximum(neg + MARGIN, 0.0)
    return softplus + sep


def _tc_body(ar, ai, pr, pi, nr, ni, out_ref):
    step = pl.program_id(0)
    a_r = ar[...]
    a_i = ai[...]
    p_r = pr[...]
    p_i = pi[...]
    n_r = nr[...]
    n_i = ni[...]

    dot_p = jnp.sum(a_r * p_r + a_i * p_i, axis=1)
    dot_n = jnp.sum(a_r * n_r + a_i * n_i, axis=1)
    ssa = jnp.sum(a_r * a_r + a_i * a_i, axis=1)
    ssp = jnp.sum(p_r * p_r + p_i * p_i, axis=1)
    ssn = jnp.sum(n_r * n_r + n_i * n_i, axis=1)

    part = jnp.sum(_row_loss(dot_p, dot_n, ssa, ssp, ssn))

    @pl.when(step == 0)
    def _():
        out_ref[0, 0] = 0.0

    out_ref[0, 0] += part


def _tail_body(x_ref, tc_ref, out_ref):
    x = x_ref[...]
    s_p = jnp.sum(x[:, 0:16], axis=1)
    s_n = jnp.sum(x[:, 16:32], axis=1)
    s_a = jnp.sum(x[:, 32:48], axis=1)
    s_pp = jnp.sum(x[:, 48:64], axis=1)
    s_nn = jnp.sum(x[:, 64:80], axis=1)
    sc_sum = jnp.sum(_row_loss(s_p, s_n, s_a, s_pp, s_nn))
    out_ref[0, 0] = (sc_sum + tc_ref[0, 0]) * (1.0 / N)


def kernel(anchor_real, anchor_imag, positive_real, positive_imag, negative_real, negative_imag):
    full = (anchor_real, anchor_imag, positive_real, positive_imag,
            negative_real, negative_imag)

    sc_sums = _sc_sums(*full)

    tc_spec = pl.BlockSpec((BN, D), lambda i: (i, 0))
    tc_part = pl.pallas_call(
        _tc_body,
        grid=(TC_N // BN,),
        in_specs=[tc_spec] * 6,
        out_specs=pl.BlockSpec(memory_space=pltpu.SMEM),
        out_shape=jax.ShapeDtypeStruct((1, 1), jnp.float32),
        compiler_params=pltpu.CompilerParams(
            dimension_semantics=("arbitrary",),
        ),
    )(*full)

    out = pl.pallas_call(
        _tail_body,
        in_specs=[
            pl.BlockSpec((SC_N, 80), lambda: (0, 0)),
            pl.BlockSpec(memory_space=pltpu.SMEM),
        ],
        out_specs=pl.BlockSpec(memory_space=pltpu.SMEM),
        out_shape=jax.ShapeDtypeStruct((1, 1), jnp.float32),
    )(sc_sums, tc_part)
    return out[0, 0]


# speedup vs baseline: 1.0307x; 1.0189x over previous
import functools
import jax
import jax.numpy as jnp
from jax import lax
from jax.experimental import pallas as pl
from jax.experimental.pallas import tpu as pltpu
from jax.experimental.pallas import tpu_sc as plsc

N, D = 16384, 1024
SC_N = 1024
TC_N = N - SC_N
NW = 32
ROWS_PER_W = SC_N // NW
GROUP = 4
NGROUPS = ROWS_PER_W // GROUP
NBUF = 2
NCHUNK = D // 16
UNROLL = 2
BN = 512
TEMP = 0.1
MARGIN = 1.0

_mesh = plsc.VectorSubcoreMesh(
    core_axis_name="c", subcore_axis_name="s", num_cores=2, num_subcores=16
)


@functools.partial(
    pl.kernel,
    out_type=jax.ShapeDtypeStruct((SC_N, 80), jnp.float32),
    mesh=_mesh,
    scratch_types=[
        pltpu.VMEM((NBUF, 6, GROUP, D), jnp.float32),
        pltpu.VMEM((GROUP, 80), jnp.float32),
        pltpu.SemaphoreType.DMA,
        pltpu.SemaphoreType.DMA,
    ],
)
def _sc_sums(ar, ai, pr, pi, nr, ni, out, buf, stage, sem0, sem1):
    sems = (sem0, sem1)
    wid = lax.axis_index("s") * 2 + lax.axis_index("c")
    base = TC_N + wid * ROWS_PER_W
    inputs = (ar, ai, pr, pi, nr, ni)

    def issue(g, b):
        r0 = base + g * GROUP
        for k in range(6):
            pltpu.async_copy(inputs[k].at[pl.ds(r0, GROUP), :], buf.at[b, k], sems[b])

    def wait_group(g, b):
        r0 = base + g * GROUP
        for k in range(6):
            pltpu.make_async_copy(
                inputs[k].at[pl.ds(r0, GROUP), :], buf.at[b, k], sems[b]
            ).wait()

    def compute_group(g, b):
        for r in range(GROUP):
            def col_body(j, accs):
                accs = list(accs)
                for u in range(UNROLL):
                    sl = pl.ds((j * UNROLL + u) * 16, 16)
                    v_ar = buf[b, 0, r, sl]
                    v_ai = buf[b, 1, r, sl]
                    v_pr = buf[b, 2, r, sl]
                    v_pi = buf[b, 3, r, sl]
                    v_nr = buf[b, 4, r, sl]
                    v_ni = buf[b, 5, r, sl]
                    prods = (v_ar * v_pr, v_ai * v_pi,
                             v_ar * v_nr, v_ai * v_ni,
                             v_ar * v_ar, v_ai * v_ai,
                             v_pr * v_pr, v_pi * v_pi,
                             v_nr * v_nr, v_ni * v_ni)
                    for q in range(10):
                        accs[q] = accs[q] + prods[q]
                return tuple(accs)

            z = jnp.zeros((16,), jnp.float32)
            accs = lax.fori_loop(0, NCHUNK // UNROLL, col_body, (z,) * 10)
            for q in range(5):
                stage[r, pl.ds(q * 16, 16)] = accs[2 * q] + accs[2 * q + 1]
        r0 = wid * ROWS_PER_W + g * GROUP
        pltpu.sync_copy(stage, out.at[pl.ds(r0, GROUP), :])

    issue(0, 0)
    issue(1, 1)

    @pl.loop(0, NGROUPS, step=NBUF)
    def _(g0):
        for b in range(NBUF):
            g = g0 + b
            wait_group(g, b)
            compute_group(g, b)

            @pl.when(g + NBUF < NGROUPS)
            def _():
                issue(g + NBUF, b)


def _row_loss(dot_p, dot_n, ssa, ssp, ssn):
    mag_a = jnp.sqrt(ssa + 1e-8)
    mag_p = jnp.sqrt(ssp + 1e-8)
    mag_n = jnp.sqrt(ssn + 1e-8)
    pos = dot_p / (mag_a * mag_p + 1e-8)
    neg = dot_n / (mag_a * mag_n + 1e-8)
    t = (neg - pos) / TEMP
    softplus = jnp.maximum(t, 0.0) + jnp.log1p(jnp.exp(-jnp.abs(t)))
    sep = jnp.maximum(neg + MARGIN, 0.0)
    return softplus + sep


def _tc_body(ar, ai, pr, pi, nr, ni, out_ref):
    step = pl.program_id(0)
    a_r = ar[...]
    a_i = ai[...]
    p_r = pr[...]
    p_i = pi[...]
    n_r = nr[...]
    n_i = ni[...]

    dot_p = jnp.sum(a_r * p_r + a_i * p_i, axis=1)
    dot_n = jnp.sum(a_r * n_r + a_i * n_i, axis=1)
    ssa = jnp.sum(a_r * a_r + a_i * a_i, axis=1)
    ssp = jnp.sum(p_r * p_r + p_i * p_i, axis=1)
    ssn = jnp.sum(n_r * n_r + n_i * n_i, axis=1)

    part = jnp.sum(_row_loss(dot_p, dot_n, ssa, ssp, ssn))

    @pl.when(step == 0)
    def _():
        out_ref[0, 0] = 0.0

    out_ref[0, 0] += part


def _tail_body(x_ref, tc_ref, out_ref):
    x = x_ref[...]
    s_p = jnp.sum(x[:, 0:16], axis=1)
    s_n = jnp.sum(x[:, 16:32], axis=1)
    s_a = jnp.sum(x[:, 32:48], axis=1)
    s_pp = jnp.sum(x[:, 48:64], axis=1)
    s_nn = jnp.sum(x[:, 64:80], axis=1)
    sc_sum = jnp.sum(_row_loss(s_p, s_n, s_a, s_pp, s_nn))
    out_ref[0, 0] = (sc_sum + tc_ref[0, 0]) * (1.0 / N)


def kernel(anchor_real, anchor_imag, positive_real, positive_imag, negative_real, negative_imag):
    full = (anchor_real, anchor_imag, positive_real, positive_imag,
            negative_real, negative_imag)

    sc_sums = _sc_sums(*full)

    tc_spec = pl.BlockSpec((BN, D), lambda i: (i, 0))
    tc_part = pl.pallas_call(
        _tc_body,
        grid=(TC_N // BN,),
        in_specs=[tc_spec] * 6,
        out_specs=pl.BlockSpec(memory_space=pltpu.SMEM),
        out_shape=jax.ShapeDtypeStruct((1, 1), jnp.float32),
        compiler_params=pltpu.CompilerParams(
            dimension_semantics=("arbitrary",),
        ),
    )(*full)

    out = pl.pallas_call(
        _tail_body,
        in_specs=[
            pl.BlockSpec((SC_N, 80), lambda: (0, 0)),
            pl.BlockSpec(memory_space=pltpu.SMEM),
        ],
        out_specs=pl.BlockSpec(memory_space=pltpu.SMEM),
        out_shape=jax.ShapeDtypeStruct((1, 1), jnp.float32),
    )(sc_sums, tc_part)
    return out[0, 0]


# speedup vs baseline: 1.0486x; 1.0173x over previous
import functools
import jax
import jax.numpy as jnp
from jax import lax
from jax.experimental import pallas as pl
from jax.experimental.pallas import tpu as pltpu
from jax.experimental.pallas import tpu_sc as plsc

N, D = 16384, 1024
SC_N = 1024
TC_N = N - SC_N
NW = 32
ROWS_PER_W = SC_N // NW
GROUP = 4
NGROUPS = ROWS_PER_W // GROUP
NBUF = 2
NCHUNK = D // 16
UNROLL = 2
BN = 512
TEMP = 0.1
MARGIN = 1.0

_mesh = plsc.VectorSubcoreMesh(
    core_axis_name="c", subcore_axis_name="s", num_cores=2, num_subcores=16
)


def _rsqrt_newton(x):
    i = lax.bitcast_convert_type(x, jnp.int32)
    i = 0x5F3759DF - lax.shift_right_logical(i, 1)
    y = lax.bitcast_convert_type(i, jnp.float32)
    for _ in range(3):
        y = y * (1.5 - 0.5 * x * y * y)
    return y


def _log1p_series(z):
    u = z / (2.0 + z)
    u2 = u * u
    return 2.0 * u * (1.0 + u2 * (1.0 / 3.0 + u2 * (0.2 + u2 * (1.0 / 7.0 + u2 * (1.0 / 9.0)))))


@functools.partial(
    pl.kernel,
    out_type=jax.ShapeDtypeStruct((NW, 16), jnp.float32),
    mesh=_mesh,
    scratch_types=[
        pltpu.VMEM((NBUF, 6, GROUP, D), jnp.float32),
        pltpu.VMEM((ROWS_PER_W, 80), jnp.float32),
        pltpu.VMEM((16,), jnp.float32),
        pltpu.SemaphoreType.DMA,
        pltpu.SemaphoreType.DMA,
    ],
)
def _sc_loss(ar, ai, pr, pi, nr, ni, out, buf, sums_v, loss_v, sem0, sem1):
    sems = (sem0, sem1)
    wid = lax.axis_index("s") * 2 + lax.axis_index("c")
    base = TC_N + wid * ROWS_PER_W
    inputs = (ar, ai, pr, pi, nr, ni)

    def issue(g, b):
        r0 = base + g * GROUP
        for k in range(6):
            pltpu.async_copy(inputs[k].at[pl.ds(r0, GROUP), :], buf.at[b, k], sems[b])

    def wait_group(g, b):
        r0 = base + g * GROUP
        for k in range(6):
            pltpu.make_async_copy(
                inputs[k].at[pl.ds(r0, GROUP), :], buf.at[b, k], sems[b]
            ).wait()

    def compute_group(g, b):
        for r in range(GROUP):
            def col_body(j, accs):
                accs = list(accs)
                for u in range(UNROLL):
                    sl = pl.ds((j * UNROLL + u) * 16, 16)
                    v_ar = buf[b, 0, r, sl]
                    v_ai = buf[b, 1, r, sl]
                    v_pr = buf[b, 2, r, sl]
                    v_pi = buf[b, 3, r, sl]
                    v_nr = buf[b, 4, r, sl]
                    v_ni = buf[b, 5, r, sl]
                    prods = (v_ar * v_pr, v_ai * v_pi,
                             v_ar * v_nr, v_ai * v_ni,
                             v_ar * v_ar, v_ai * v_ai,
                             v_pr * v_pr, v_pi * v_pi,
                             v_nr * v_nr, v_ni * v_ni)
                    for q in range(10):
                        accs[q] = accs[q] + prods[q]
                return tuple(accs)

            z = jnp.zeros((16,), jnp.float32)
            accs = lax.fori_loop(0, NCHUNK // UNROLL, col_body, (z,) * 10)
            row = g * GROUP + r
            for q in range(5):
                sums_v[row, pl.ds(q * 16, 16)] = accs[2 * q] + accs[2 * q + 1]

    issue(0, 0)
    issue(1, 1)

    @pl.loop(0, NGROUPS, step=NBUF)
    def _(g0):
        for b in range(NBUF):
            g = g0 + b
            wait_group(g, b)
            compute_group(g, b)

            @pl.when(g + NBUF < NGROUPS)
            def _():
                issue(g + NBUF, b)

    @pl.loop(0, ROWS_PER_W, init_carry=jnp.zeros((16,), jnp.float32))
    def loss_loop(r, loss_acc):
        s = []
        for q in range(5):
            v = sums_v[r, pl.ds(q * 16, 16)]
            tot = v[0]
            for l in range(1, 16):
                tot = tot + v[l]
            s.append(jnp.full((16,), tot, jnp.float32))
        dot_p, dot_n, ssa, ssp, ssn = s

        sa = ssa + 1e-8
        sp = ssp + 1e-8
        sn = ssn + 1e-8
        mag_a = sa * _rsqrt_newton(sa)
        mag_p = sp * _rsqrt_newton(sp)
        mag_n = sn * _rsqrt_newton(sn)
        pos = dot_p / (mag_a * mag_p + 1e-8)
        neg = dot_n / (mag_a * mag_n + 1e-8)

        t = (neg - pos) * (1.0 / TEMP)
        e = jnp.exp(-jnp.abs(t))
        softplus = jnp.maximum(t, 0.0) + _log1p_series(e)
        sep = jnp.maximum(neg + MARGIN, 0.0)
        return loss_acc + softplus + sep

    loss_v[...] = loss_loop * (1.0 / 16.0)
    pltpu.sync_copy(loss_v, out.at[wid])


def _row_loss(dot_p, dot_n, ssa, ssp, ssn):
    mag_a = jnp.sqrt(ssa + 1e-8)
    mag_p = jnp.sqrt(ssp + 1e-8)
    mag_n = jnp.sqrt(ssn + 1e-8)
    pos = dot_p / (mag_a * mag_p + 1e-8)
    neg = dot_n / (mag_a * mag_n + 1e-8)
    t = (neg - pos) / TEMP
    softplus = jnp.maximum(t, 0.0) + jnp.log1p(jnp.exp(-jnp.abs(t)))
    sep = jnp.maximum(neg + MARGIN, 0.0)
    return softplus + sep


def _tc_body(ar, ai, pr, pi, nr, ni, out_ref):
    step = pl.program_id(0)
    a_r = ar[...]
    a_i = ai[...]
    p_r = pr[...]
    p_i = pi[...]
    n_r = nr[...]
    n_i = ni[...]

    dot_p = jnp.sum(a_r * p_r + a_i * p_i, axis=1)
    dot_n = jnp.sum(a_r * n_r + a_i * n_i, axis=1)
    ssa = jnp.sum(a_r * a_r + a_i * a_i, axis=1)
    ssp = jnp.sum(p_r * p_r + p_i * p_i, axis=1)
    ssn = jnp.sum(n_r * n_r + n_i * n_i, axis=1)

    part = jnp.sum(_row_loss(dot_p, dot_n, ssa, ssp, ssn))

    @pl.when(step == 0)
    def _():
        out_ref[0, 0] = 0.0

    out_ref[0, 0] += part


def kernel(anchor_real, anchor_imag, positive_real, positive_imag, negative_real, negative_imag):
    full = (anchor_real, anchor_imag, positive_real, positive_imag,
            negative_real, negative_imag)

    sc_part = _sc_loss(*full)

    tc_spec = pl.BlockSpec((BN, D), lambda i: (i, 0))
    tc_part = pl.pallas_call(
        _tc_body,
        grid=(TC_N // BN,),
        in_specs=[tc_spec] * 6,
        out_specs=pl.BlockSpec(memory_space=pltpu.SMEM),
        out_shape=jax.ShapeDtypeStruct((1, 1), jnp.float32),
        compiler_params=pltpu.CompilerParams(
            dimension_semantics=("arbitrary",),
        ),
    )(*full)

    return (tc_part[0, 0] + jnp.sum(sc_part)) * (1.0 / N)
